# baseline (speedup 1.0000x reference)
import math

import jax
import jax.numpy as jnp
from jax import lax
from jax.experimental import pallas as pl
from jax.experimental.pallas import tpu as pltpu
from jax.experimental.pallas import tpu_sc as plsc

B, N, D, HF, WF = 1, 6, 48, 32, 44
CCTX = 80
BEVH, BEVW, BEVZ, BEVC = 128, 128, 8, 128
STRIDE = 4
PC = (-50.0, -50.0, -5.0, 50.0, 50.0, 3.0)
PIX = HF * WF
NPIX = N * PIX
P = NPIX * D
HW = BEVH * BEVW
INVALID = BEVZ * HW
FW = 128

NTILES = 16
TPTS = P // NTILES
NCHUNK = 6
CH = TPTS // NCHUNK
SUBB = 64
RING = 4
LOOK = 2
SLAB = HW // 2
GARB = 64
AROWS = SLAB + GARB
ZSTR = AROWS // NTILES
CSTR = SLAB // NTILES


def _geometry(intrinsics, cam2ego):
    b, n, d, hf, wf = B, N, D, HF, WF
    xs = (jnp.arange(wf, dtype=jnp.float32) + 0.5) * STRIDE
    ys = (jnp.arange(hf, dtype=jnp.float32) + 0.5) * STRIDE
    v, u = jnp.meshgrid(ys, xs, indexing='ij')
    u = u.reshape(1, 1, 1, hf, wf)
    v = v.reshape(1, 1, 1, hf, wf)
    Z = jnp.linspace(1.0, 60.0, d).reshape(1, 1, d, 1, 1)
    fx = intrinsics[:, :, 0, 0].reshape(b, n, 1, 1, 1)
    fy = intrinsics[:, :, 1, 1].reshape(b, n, 1, 1, 1)
    cx = intrinsics[:, :, 0, 2].reshape(b, n, 1, 1, 1)
    cy = intrinsics[:, :, 1, 2].reshape(b, n, 1, 1, 1)
    Xc = (u - cx) / fx * Z
    Yc = (v - cy) / fy * Z
    Zc = jnp.broadcast_to(Z, Xc.shape)
    pts = jnp.stack([Xc, Yc, Zc, jnp.ones_like(Xc)], axis=-1)
    T = cam2ego.reshape(b, n, 1, 1, 1, 4, 4)
    pe = jnp.matmul(T, pts[..., None])[..., 0][..., :3]
    x_min, y_min, z_min, x_max, y_max, z_max = PC
    mx = (x_max - x_min) / BEVW
    my = (y_max - y_min) / BEVH
    mz = (z_max - z_min) / BEVZ
    ix = jnp.floor((pe[..., 0] - x_min) / mx).astype(jnp.int32)
    iy = jnp.floor((pe[..., 1] - y_min) / my).astype(jnp.int32)
    iz = jnp.floor((pe[..., 2] - z_min) / mz).astype(jnp.int32)
    valid = ((ix >= 0) & (ix < BEVW) & (iy >= 0) & (iy < BEVH)
             & (iz >= 0) & (iz < BEVZ))
    vind = (iz * BEVH + iy) * BEVW + ix
    return jnp.where(valid, vind, INVALID)


DBLK = 8


def _feat_body(dp_ref, ctx_ref, o_ref):
    c = ctx_ref[0]
    ones = jnp.ones((PIX, 1), jnp.float32)
    zeros = jnp.zeros((PIX, FW - CCTX - 1), jnp.float32)
    for d in range(DBLK):
        prod = dp_ref[0, d] * c
        o_ref[0, d] = jnp.concatenate([prod, ones, zeros], axis=1)


def _feat(dp4, ctx3, interpret=False):
    out = pl.pallas_call(
        _feat_body,
        grid=(N, D // DBLK),
        in_specs=[
            pl.BlockSpec((1, DBLK, PIX, 1), lambda i, j: (i, j, 0, 0)),
            pl.BlockSpec((1, PIX, CCTX), lambda i, j: (i, 0, 0)),
        ],
        out_specs=pl.BlockSpec((1, DBLK, PIX, FW), lambda i, j: (i, j, 0, 0)),
        out_shape=jax.ShapeDtypeStruct((N, D, PIX, FW), jnp.float32),
        interpret=interpret,
    )(dp4, ctx3)
    return out.reshape(P, FW)


def _sc_body(feat_hbm, vind_hbm, vox_hbm,
             A, vbuf, ptl, hwl, ptrow, hwrow, fstage, zbuf, sem_g, sem_s):
    core = lax.axis_index("c")
    sub = lax.axis_index("s")
    iota16 = jnp.arange(16, dtype=jnp.int32)

    def _zb(i, _):
        for cc in range(8):
            zbuf[i, pl.ds(cc * 16, 16)] = jnp.zeros((16,), jnp.float32)
        return 0
    lax.fori_loop(0, ZSTR // 12, _zb, 0)

    def _pass(p, _):
        kcur = p * 2 + core

        for q in range(12):
            row0 = sub * ZSTR + q * (ZSTR // 12)
            pltpu.sync_copy(zbuf, A.at[pl.ds(row0, ZSTR // 12)])
        plsc.subcore_barrier()

        def _chunk(kc, _):
            base = sub * TPTS + kc * CH
            pltpu.sync_copy(vind_hbm.at[pl.ds(base, CH)], vbuf)

            def _filt(g, cur):
                vv = vbuf[pl.ds(g * 16, 16)]
                m = (vv >> 13) == kcur
                mi = m.astype(jnp.int32)
                cnt = jnp.sum(mi)

                @pl.when(cnt > 0)
                def _():
                    hwv = vv & (SLAB - 1)
                    ptv = base + g * 16 + iota16
                    incl = plsc.cumsum(mi)
                    pos = cur + incl - mi
                    plsc.store_scatter(hwl, [pos], hwv, mask=m)
                    plsc.store_scatter(ptl, [pos], ptv, mask=m)
                return cur + cnt
            nsel = lax.fori_loop(0, CH // 16, _filt, 0)

            def _pad(g, cur):
                pos = cur + iota16
                plsc.store_scatter(
                    hwl, [pos], SLAB + ((g * 16 + iota16) & (GARB - 1)))
                plsc.store_scatter(ptl, [pos], g * 16 + iota16)
                return cur + 16
            lax.fori_loop(0, SUBB // 16, _pad, nsel)

            nb = (nsel + SUBB - 1) >> 6

            def _fire(j):
                slot = lax.rem(j, RING)
                for r in range(SUBB // 16):
                    ptrow[slot, pl.ds(r * 16, 16)] = \
                        ptl[pl.ds(j * SUBB + r * 16, 16)]
                pltpu.async_copy(feat_hbm.at[ptrow.at[slot]], fstage.at[slot],
                                 sem_g)

            def _pro(j, _):
                @pl.when(j < nb)
                def _():
                    _fire(j)
                return 0
            lax.fori_loop(0, LOOK, _pro, 0)

            def _batch(j, _):
                slot = lax.rem(j, RING)

                @pl.when(j >= LOOK)
                def _():
                    sl2 = lax.rem(j - LOOK, RING)
                    pltpu.make_async_copy(fstage.at[sl2], A.at[hwrow.at[sl2]],
                                          sem_s).wait()

                @pl.when(j + LOOK < nb)
                def _():
                    _fire(j + LOOK)

                pltpu.make_async_copy(feat_hbm.at[pl.ds(0, SUBB)],
                                      fstage.at[slot], sem_g).wait()
                for r in range(SUBB // 16):
                    hwrow[slot, pl.ds(r * 16, 16)] = \
                        hwl[pl.ds(j * SUBB + r * 16, 16)]
                pltpu.async_copy(fstage.at[slot], A.at[hwrow.at[slot]], sem_s,
                                 add=True)
                return 0
            lax.fori_loop(0, nb, _batch, 0)

            def _sdrain(j, _):
                slot = lax.rem(j, RING)
                pltpu.make_async_copy(fstage.at[slot], A.at[hwrow.at[slot]],
                                      sem_s).wait()
                return 0
            lax.fori_loop(jnp.maximum(nb - LOOK, 0), nb, _sdrain, 0)
            return 0
        lax.fori_loop(0, NCHUNK, _chunk, 0)
        plsc.subcore_barrier()

        vbase = kcur * SLAB + sub * CSTR
        pltpu.sync_copy(A.at[pl.ds(sub * CSTR, CSTR)],
                        vox_hbm.at[pl.ds(vbase, CSTR)])
        plsc.subcore_barrier()
        return 0
    lax.fori_loop(0, 8, _pass, 0)


def _sc_scatter(feat, vind):
    mesh = plsc.VectorSubcoreMesh(core_axis_name="c", subcore_axis_name="s",
                                  num_cores=2, num_subcores=NTILES)
    f = pl.kernel(
        _sc_body,
        out_type=jax.ShapeDtypeStruct((BEVZ * HW, FW), jnp.float32),
        mesh=mesh,
        compiler_params=pltpu.CompilerParams(needs_layout_passes=False),
        scratch_types=[
            pltpu.VMEM_SHARED((AROWS, FW), jnp.float32),
            pltpu.VMEM((CH,), jnp.int32),
            pltpu.VMEM((CH + SUBB,), jnp.int32),
            pltpu.VMEM((CH + SUBB,), jnp.int32),
            pltpu.VMEM((RING, SUBB), jnp.int32),
            pltpu.VMEM((RING, SUBB), jnp.int32),
            pltpu.VMEM((RING, SUBB, FW), jnp.float32),
            pltpu.VMEM((ZSTR // 12, FW), jnp.float32),
            pltpu.SemaphoreType.DMA,
            pltpu.SemaphoreType.DMA,
        ],
    )
    return f(feat, vind)


def _finish_body(wr_ref, vox_ref, gam_ref, bet_ref, o_ref):
    acc = jnp.zeros((BEVC, 128), jnp.float32)
    for z in range(BEVZ):
        blk = vox_ref[z]
        vb = blk[:, :CCTX]
        c = blk[:, CCTX]
        inv = 1.0 / jnp.maximum(c, 1.0)
        vbn = vb * inv[:, None]
        acc += lax.dot_general(wr_ref[z], vbn, (((1,), (1,)), ((), ())),
                               preferred_element_type=jnp.float32)
    o_ref[...] = jnp.maximum(acc * gam_ref[...] + bet_ref[...], 0.0)


def _finish(wr, vox, gs, beta, interpret=False):
    grid = HW // 128
    return pl.pallas_call(
        _finish_body,
        grid=(grid,),
        in_specs=[
            pl.BlockSpec((BEVZ, BEVC, CCTX), lambda i: (0, 0, 0)),
            pl.BlockSpec((BEVZ, 128, FW), lambda i: (0, i, 0)),
            pl.BlockSpec((BEVC, 1), lambda i: (0, 0)),
            pl.BlockSpec((BEVC, 1), lambda i: (0, 0)),
        ],
        out_specs=pl.BlockSpec((BEVC, 128), lambda i: (0, i)),
        out_shape=jax.ShapeDtypeStruct((BEVC, HW), jnp.float32),
        interpret=interpret,
    )(wr, vox, gs, beta)


def kernel(depth_prob, context, intrinsics, cam2ego, W, gamma, beta):
    vind = _geometry(intrinsics, cam2ego).reshape(-1)
    dp4 = depth_prob.reshape(N, D, PIX, 1)
    ctx3 = jnp.transpose(context.reshape(N, CCTX, PIX), (0, 2, 1))
    feat = _feat(dp4, ctx3)
    vox = _sc_scatter(feat, vind)
    vox3 = vox.reshape(BEVZ, HW, FW)
    wr = jnp.transpose(W.reshape(BEVC, CCTX, BEVZ), (2, 0, 1))
    gs = (gamma / math.sqrt(1.0 + 1e-5)).reshape(BEVC, 1)
    y = _finish(wr, vox3, gs, beta.reshape(BEVC, 1))
    return y.reshape(1, BEVC, BEVH, BEVW)


# speedup vs baseline: 1.0950x; 1.0950x over previous
import math

import jax
import jax.numpy as jnp
from jax import lax
from jax.experimental import pallas as pl
from jax.experimental.pallas import tpu as pltpu
from jax.experimental.pallas import tpu_sc as plsc

B, N, D, HF, WF = 1, 6, 48, 32, 44
CCTX = 80
BEVH, BEVW, BEVZ, BEVC = 128, 128, 8, 128
STRIDE = 4
PC = (-50.0, -50.0, -5.0, 50.0, 50.0, 3.0)
PIX = HF * WF
NPIX = N * PIX
P = NPIX * D
HW = BEVH * BEVW
INVALID = BEVZ * HW
FW = 128

NTILES = 16
TPTS = P // NTILES
NCHUNK = 6
CH = TPTS // NCHUNK
SUBB = 128
RING = 2
LOOK = 1
SLAB = HW // 2
GARB = 64
AROWS = SLAB + GARB
ZSTR = AROWS // NTILES
CSTR = SLAB // NTILES


def _geometry(intrinsics, cam2ego):
    b, n, d, hf, wf = B, N, D, HF, WF
    xs = (jnp.arange(wf, dtype=jnp.float32) + 0.5) * STRIDE
    ys = (jnp.arange(hf, dtype=jnp.float32) + 0.5) * STRIDE
    v, u = jnp.meshgrid(ys, xs, indexing='ij')
    u = u.reshape(1, 1, 1, hf, wf)
    v = v.reshape(1, 1, 1, hf, wf)
    Z = jnp.linspace(1.0, 60.0, d).reshape(1, 1, d, 1, 1)
    fx = intrinsics[:, :, 0, 0].reshape(b, n, 1, 1, 1)
    fy = intrinsics[:, :, 1, 1].reshape(b, n, 1, 1, 1)
    cx = intrinsics[:, :, 0, 2].reshape(b, n, 1, 1, 1)
    cy = intrinsics[:, :, 1, 2].reshape(b, n, 1, 1, 1)
    Xc = (u - cx) / fx * Z
    Yc = (v - cy) / fy * Z
    Zc = jnp.broadcast_to(Z, Xc.shape)
    pts = jnp.stack([Xc, Yc, Zc, jnp.ones_like(Xc)], axis=-1)
    T = cam2ego.reshape(b, n, 1, 1, 1, 4, 4)
    pe = jnp.matmul(T, pts[..., None])[..., 0][..., :3]
    x_min, y_min, z_min, x_max, y_max, z_max = PC
    mx = (x_max - x_min) / BEVW
    my = (y_max - y_min) / BEVH
    mz = (z_max - z_min) / BEVZ
    ix = jnp.floor((pe[..., 0] - x_min) / mx).astype(jnp.int32)
    iy = jnp.floor((pe[..., 1] - y_min) / my).astype(jnp.int32)
    iz = jnp.floor((pe[..., 2] - z_min) / mz).astype(jnp.int32)
    valid = ((ix >= 0) & (ix < BEVW) & (iy >= 0) & (iy < BEVH)
             & (iz >= 0) & (iz < BEVZ))
    vind = (iz * BEVH + iy) * BEVW + ix
    return jnp.where(valid, vind, INVALID)


DBLK = 8


def _feat_body(dp_ref, ctx_ref, o_ref):
    c = ctx_ref[0]
    ones = jnp.ones((PIX, 1), jnp.float32)
    zeros = jnp.zeros((PIX, FW - CCTX - 1), jnp.float32)
    for d in range(DBLK):
        prod = dp_ref[0, d] * c
        o_ref[0, d] = jnp.concatenate([prod, ones, zeros], axis=1)


def _feat(dp4, ctx3, interpret=False):
    out = pl.pallas_call(
        _feat_body,
        grid=(N, D // DBLK),
        in_specs=[
            pl.BlockSpec((1, DBLK, PIX, 1), lambda i, j: (i, j, 0, 0)),
            pl.BlockSpec((1, PIX, CCTX), lambda i, j: (i, 0, 0)),
        ],
        out_specs=pl.BlockSpec((1, DBLK, PIX, FW), lambda i, j: (i, j, 0, 0)),
        out_shape=jax.ShapeDtypeStruct((N, D, PIX, FW), jnp.float32),
        interpret=interpret,
    )(dp4, ctx3)
    return out.reshape(P, FW)


def _sc_body(feat_hbm, vind_hbm, vox_hbm,
             A, vbuf, ptl, hwl, ptrow, hwrow, fstage, zbuf, sem_g, sem_s):
    core = lax.axis_index("c")
    sub = lax.axis_index("s")
    iota16 = jnp.arange(16, dtype=jnp.int32)

    def _zb(i, _):
        for cc in range(8):
            zbuf[i, pl.ds(cc * 16, 16)] = jnp.zeros((16,), jnp.float32)
        return 0
    lax.fori_loop(0, ZSTR // 12, _zb, 0)

    def _pass(p, _):
        kcur = p * 2 + core

        for q in range(12):
            row0 = sub * ZSTR + q * (ZSTR // 12)
            pltpu.sync_copy(zbuf, A.at[pl.ds(row0, ZSTR // 12)])
        plsc.subcore_barrier()

        def _chunk(kc, _):
            base = sub * TPTS + kc * CH
            pltpu.sync_copy(vind_hbm.at[pl.ds(base, CH)], vbuf)

            def _filt(g, cur):
                vv = vbuf[pl.ds(g * 16, 16)]
                m = (vv >> 13) == kcur
                hwv = vv & (SLAB - 1)
                ptv = base + g * 16 + iota16
                mi = m.astype(jnp.int32)
                incl = plsc.cumsum(mi)
                pos = cur + incl - mi
                plsc.store_scatter(hwl, [pos], hwv, mask=m)
                plsc.store_scatter(ptl, [pos], ptv, mask=m)
                return cur + jnp.sum(mi)
            nsel = lax.fori_loop(0, CH // 16, _filt, 0)

            def _pad(g, cur):
                pos = cur + iota16
                plsc.store_scatter(
                    hwl, [pos], SLAB + ((g * 16 + iota16) & (GARB - 1)))
                plsc.store_scatter(ptl, [pos], g * 16 + iota16)
                return cur + 16
            lax.fori_loop(0, SUBB // 16, _pad, nsel)

            nb = (nsel + SUBB - 1) >> 7

            def _fire(j):
                slot = lax.rem(j, RING)
                for r in range(SUBB // 16):
                    ptrow[slot, pl.ds(r * 16, 16)] = \
                        ptl[pl.ds(j * SUBB + r * 16, 16)]
                pltpu.async_copy(feat_hbm.at[ptrow.at[slot]], fstage.at[slot],
                                 sem_g)

            def _pro(j, _):
                @pl.when(j < nb)
                def _():
                    _fire(j)
                return 0
            lax.fori_loop(0, LOOK, _pro, 0)

            def _batch(j, _):
                slot = lax.rem(j, RING)

                @pl.when(j >= LOOK)
                def _():
                    sl2 = lax.rem(j - LOOK, RING)
                    pltpu.make_async_copy(fstage.at[sl2], A.at[hwrow.at[sl2]],
                                          sem_s).wait()

                @pl.when(j + LOOK < nb)
                def _():
                    _fire(j + LOOK)

                pltpu.make_async_copy(feat_hbm.at[pl.ds(0, SUBB)],
                                      fstage.at[slot], sem_g).wait()
                for r in range(SUBB // 16):
                    hwrow[slot, pl.ds(r * 16, 16)] = \
                        hwl[pl.ds(j * SUBB + r * 16, 16)]
                pltpu.async_copy(fstage.at[slot], A.at[hwrow.at[slot]], sem_s,
                                 add=True)
                return 0
            lax.fori_loop(0, nb, _batch, 0)

            def _sdrain(j, _):
                slot = lax.rem(j, RING)
                pltpu.make_async_copy(fstage.at[slot], A.at[hwrow.at[slot]],
                                      sem_s).wait()
                return 0
            lax.fori_loop(jnp.maximum(nb - LOOK, 0), nb, _sdrain, 0)
            return 0
        lax.fori_loop(0, NCHUNK, _chunk, 0)
        plsc.subcore_barrier()

        vbase = kcur * SLAB + sub * CSTR
        pltpu.sync_copy(A.at[pl.ds(sub * CSTR, CSTR)],
                        vox_hbm.at[pl.ds(vbase, CSTR)])
        plsc.subcore_barrier()
        return 0
    lax.fori_loop(0, 8, _pass, 0)


def _sc_scatter(feat, vind):
    mesh = plsc.VectorSubcoreMesh(core_axis_name="c", subcore_axis_name="s",
                                  num_cores=2, num_subcores=NTILES)
    f = pl.kernel(
        _sc_body,
        out_type=jax.ShapeDtypeStruct((BEVZ * HW, FW), jnp.float32),
        mesh=mesh,
        compiler_params=pltpu.CompilerParams(needs_layout_passes=False),
        scratch_types=[
            pltpu.VMEM_SHARED((AROWS, FW), jnp.float32),
            pltpu.VMEM((CH,), jnp.int32),
            pltpu.VMEM((CH + SUBB,), jnp.int32),
            pltpu.VMEM((CH + SUBB,), jnp.int32),
            pltpu.VMEM((RING, SUBB), jnp.int32),
            pltpu.VMEM((RING, SUBB), jnp.int32),
            pltpu.VMEM((RING, SUBB, FW), jnp.float32),
            pltpu.VMEM((ZSTR // 12, FW), jnp.float32),
            pltpu.SemaphoreType.DMA,
            pltpu.SemaphoreType.DMA,
        ],
    )
    return f(feat, vind)


def _finish_body(wr_ref, vox_ref, gam_ref, bet_ref, o_ref):
    acc = jnp.zeros((BEVC, 128), jnp.float32)
    for z in range(BEVZ):
        blk = vox_ref[z]
        vb = blk[:, :CCTX]
        c = blk[:, CCTX]
        inv = 1.0 / jnp.maximum(c, 1.0)
        vbn = vb * inv[:, None]
        acc += lax.dot_general(wr_ref[z], vbn, (((1,), (1,)), ((), ())),
                               preferred_element_type=jnp.float32)
    o_ref[...] = jnp.maximum(acc * gam_ref[...] + bet_ref[...], 0.0)


def _finish(wr, vox, gs, beta, interpret=False):
    grid = HW // 128
    return pl.pallas_call(
        _finish_body,
        grid=(grid,),
        in_specs=[
            pl.BlockSpec((BEVZ, BEVC, CCTX), lambda i: (0, 0, 0)),
            pl.BlockSpec((BEVZ, 128, FW), lambda i: (0, i, 0)),
            pl.BlockSpec((BEVC, 1), lambda i: (0, 0)),
            pl.BlockSpec((BEVC, 1), lambda i: (0, 0)),
        ],
        out_specs=pl.BlockSpec((BEVC, 128), lambda i: (0, i)),
        out_shape=jax.ShapeDtypeStruct((BEVC, HW), jnp.float32),
        interpret=interpret,
    )(wr, vox, gs, beta)


def kernel(depth_prob, context, intrinsics, cam2ego, W, gamma, beta):
    vind = _geometry(intrinsics, cam2ego).reshape(-1)
    dp4 = depth_prob.reshape(N, D, PIX, 1)
    ctx3 = jnp.transpose(context.reshape(N, CCTX, PIX), (0, 2, 1))
    feat = _feat(dp4, ctx3)
    vox = _sc_scatter(feat, vind)
    vox3 = vox.reshape(BEVZ, HW, FW)
    wr = jnp.transpose(W.reshape(BEVC, CCTX, BEVZ), (2, 0, 1))
    gs = (gamma / math.sqrt(1.0 + 1e-5)).reshape(BEVC, 1)
    y = _finish(wr, vox3, gs, beta.reshape(BEVC, 1))
    return y.reshape(1, BEVC, BEVH, BEVW)


# speedup vs baseline: 1.2141x; 1.1087x over previous
import math

import jax
import jax.numpy as jnp
from jax import lax
from jax.experimental import pallas as pl
from jax.experimental.pallas import tpu as pltpu
from jax.experimental.pallas import tpu_sc as plsc

B, N, D, HF, WF = 1, 6, 48, 32, 44
CCTX = 80
BEVH, BEVW, BEVZ, BEVC = 128, 128, 8, 128
STRIDE = 4
PC = (-50.0, -50.0, -5.0, 50.0, 50.0, 3.0)
PIX = HF * WF
NPIX = N * PIX
P = NPIX * D
HW = BEVH * BEVW
INVALID = BEVZ * HW
FW = 128

NTILES = 16
TPTS = P // NTILES
NCHUNK = 4
CH = TPTS // NCHUNK
SUBB = 64
RING = 4
LOOK = 2
SLAB = HW // 2
GARB = 64
AROWS = SLAB + GARB
ZSTR = AROWS // NTILES
CSTR = SLAB // NTILES


def _geometry(intrinsics, cam2ego):
    b, n, d, hf, wf = B, N, D, HF, WF
    xs = (jnp.arange(wf, dtype=jnp.float32) + 0.5) * STRIDE
    ys = (jnp.arange(hf, dtype=jnp.float32) + 0.5) * STRIDE
    v, u = jnp.meshgrid(ys, xs, indexing='ij')
    u = u.reshape(1, 1, 1, hf, wf)
    v = v.reshape(1, 1, 1, hf, wf)
    Z = jnp.linspace(1.0, 60.0, d).reshape(1, 1, d, 1, 1)
    fx = intrinsics[:, :, 0, 0].reshape(b, n, 1, 1, 1)
    fy = intrinsics[:, :, 1, 1].reshape(b, n, 1, 1, 1)
    cx = intrinsics[:, :, 0, 2].reshape(b, n, 1, 1, 1)
    cy = intrinsics[:, :, 1, 2].reshape(b, n, 1, 1, 1)
    Xc = (u - cx) / fx * Z
    Yc = (v - cy) / fy * Z
    Zc = jnp.broadcast_to(Z, Xc.shape)
    pts = jnp.stack([Xc, Yc, Zc, jnp.ones_like(Xc)], axis=-1)
    T = cam2ego.reshape(b, n, 1, 1, 1, 4, 4)
    pe = jnp.matmul(T, pts[..., None])[..., 0][..., :3]
    x_min, y_min, z_min, x_max, y_max, z_max = PC
    mx = (x_max - x_min) / BEVW
    my = (y_max - y_min) / BEVH
    mz = (z_max - z_min) / BEVZ
    ix = jnp.floor((pe[..., 0] - x_min) / mx).astype(jnp.int32)
    iy = jnp.floor((pe[..., 1] - y_min) / my).astype(jnp.int32)
    iz = jnp.floor((pe[..., 2] - z_min) / mz).astype(jnp.int32)
    valid = ((ix >= 0) & (ix < BEVW) & (iy >= 0) & (iy < BEVH)
             & (iz >= 0) & (iz < BEVZ))
    vind = (iz * BEVH + iy) * BEVW + ix
    return jnp.where(valid, vind, INVALID)


DBLK = 8


def _feat_body(dp_ref, ctx_ref, o_ref):
    c = ctx_ref[0]
    ones = jnp.ones((PIX, 1), jnp.float32)
    zeros = jnp.zeros((PIX, FW - CCTX - 1), jnp.float32)
    for d in range(DBLK):
        prod = dp_ref[0, d] * c
        o_ref[0, d] = jnp.concatenate([prod, ones, zeros], axis=1)


def _feat(dp4, ctx3, interpret=False):
    out = pl.pallas_call(
        _feat_body,
        grid=(N, D // DBLK),
        in_specs=[
            pl.BlockSpec((1, DBLK, PIX, 1), lambda i, j: (i, j, 0, 0)),
            pl.BlockSpec((1, PIX, CCTX), lambda i, j: (i, 0, 0)),
        ],
        out_specs=pl.BlockSpec((1, DBLK, PIX, FW), lambda i, j: (i, j, 0, 0)),
        out_shape=jax.ShapeDtypeStruct((N, D, PIX, FW), jnp.float32),
        interpret=interpret,
    )(dp4, ctx3)
    return out.reshape(P, FW)


def _sc_body(feat_hbm, vind_hbm, vox_hbm,
             A, vbuf, ptl, hwl, ptrow, hwrow, fstage, zbuf, sem_g, sem_s):
    core = lax.axis_index("c")
    sub = lax.axis_index("s")
    iota16 = jnp.arange(16, dtype=jnp.int32)

    def _zb(i, _):
        for cc in range(8):
            zbuf[i, pl.ds(cc * 16, 16)] = jnp.zeros((16,), jnp.float32)
        return 0
    lax.fori_loop(0, ZSTR // 12, _zb, 0)

    def _pass(p, _):
        kcur = p * 2 + core

        for q in range(12):
            row0 = sub * ZSTR + q * (ZSTR // 12)
            pltpu.sync_copy(zbuf, A.at[pl.ds(row0, ZSTR // 12)])
        plsc.subcore_barrier()

        def _chunk(kc, _):
            base = sub * TPTS + kc * CH
            pltpu.sync_copy(vind_hbm.at[pl.ds(base, CH)], vbuf)

            def _filt(g, cur):
                vv = vbuf[pl.ds(g * 16, 16)]
                m = (vv >> 13) == kcur
                hwv = vv & (SLAB - 1)
                ptv = base + g * 16 + iota16
                mi = m.astype(jnp.int32)
                incl = plsc.cumsum(mi)
                pos = cur + incl - mi
                plsc.store_scatter(hwl, [pos], hwv, mask=m)
                plsc.store_scatter(ptl, [pos], ptv, mask=m)
                return cur + jnp.sum(mi)
            nsel = lax.fori_loop(0, CH // 16, _filt, 0)

            def _pad(g, cur):
                pos = cur + iota16
                plsc.store_scatter(
                    hwl, [pos], SLAB + ((g * 16 + iota16) & (GARB - 1)))
                plsc.store_scatter(ptl, [pos], g * 16 + iota16)
                return cur + 16
            lax.fori_loop(0, SUBB // 16, _pad, nsel)

            nb = (nsel + SUBB - 1) >> 6

            def _fire(j):
                slot = lax.rem(j, RING)
                for r in range(SUBB // 16):
                    ptrow[slot, pl.ds(r * 16, 16)] = \
                        ptl[pl.ds(j * SUBB + r * 16, 16)]
                pltpu.async_copy(feat_hbm.at[ptrow.at[slot]], fstage.at[slot],
                                 sem_g)

            def _pro(j, _):
                @pl.when(j < nb)
                def _():
                    _fire(j)
                return 0
            lax.fori_loop(0, LOOK, _pro, 0)

            def _batch(j, _):
                slot = lax.rem(j, RING)

                @pl.when(j >= LOOK)
                def _():
                    sl2 = lax.rem(j - LOOK, RING)
                    pltpu.make_async_copy(fstage.at[sl2], A.at[hwrow.at[sl2]],
                                          sem_s).wait()

                @pl.when(j + LOOK < nb)
                def _():
                    _fire(j + LOOK)

                pltpu.make_async_copy(feat_hbm.at[pl.ds(0, SUBB)],
                                      fstage.at[slot], sem_g).wait()
                for r in range(SUBB // 16):
                    hwrow[slot, pl.ds(r * 16, 16)] = \
                        hwl[pl.ds(j * SUBB + r * 16, 16)]
                pltpu.async_copy(fstage.at[slot], A.at[hwrow.at[slot]], sem_s,
                                 add=True)
                return 0
            lax.fori_loop(0, nb, _batch, 0)

            def _sdrain(j, _):
                slot = lax.rem(j, RING)
                pltpu.make_async_copy(fstage.at[slot], A.at[hwrow.at[slot]],
                                      sem_s).wait()
                return 0
            lax.fori_loop(jnp.maximum(nb - LOOK, 0), nb, _sdrain, 0)
            return 0
        lax.fori_loop(0, NCHUNK, _chunk, 0)
        plsc.subcore_barrier()

        vbase = kcur * SLAB + sub * CSTR
        pltpu.sync_copy(A.at[pl.ds(sub * CSTR, CSTR)],
                        vox_hbm.at[pl.ds(vbase, CSTR)])
        plsc.subcore_barrier()
        return 0
    lax.fori_loop(0, 8, _pass, 0)


def _sc_scatter(feat, vind):
    mesh = plsc.VectorSubcoreMesh(core_axis_name="c", subcore_axis_name="s",
                                  num_cores=2, num_subcores=NTILES)
    f = pl.kernel(
        _sc_body,
        out_type=jax.ShapeDtypeStruct((BEVZ * HW, FW), jnp.float32),
        mesh=mesh,
        compiler_params=pltpu.CompilerParams(needs_layout_passes=False),
        scratch_types=[
            pltpu.VMEM_SHARED((AROWS, FW), jnp.float32),
            pltpu.VMEM((CH,), jnp.int32),
            pltpu.VMEM((CH + SUBB,), jnp.int32),
            pltpu.VMEM((CH + SUBB,), jnp.int32),
            pltpu.VMEM((RING, SUBB), jnp.int32),
            pltpu.VMEM((RING, SUBB), jnp.int32),
            pltpu.VMEM((RING, SUBB, FW), jnp.float32),
            pltpu.VMEM((ZSTR // 12, FW), jnp.float32),
            pltpu.SemaphoreType.DMA,
            pltpu.SemaphoreType.DMA,
        ],
    )
    return f(feat, vind)


def _finish_body(wr_ref, vox_ref, gam_ref, bet_ref, o_ref):
    acc = jnp.zeros((BEVC, 128), jnp.float32)
    for z in range(BEVZ):
        blk = vox_ref[z]
        vb = blk[:, :CCTX]
        c = blk[:, CCTX]
        inv = 1.0 / jnp.maximum(c, 1.0)
        vbn = vb * inv[:, None]
        acc += lax.dot_general(wr_ref[z], vbn, (((1,), (1,)), ((), ())),
                               preferred_element_type=jnp.float32)
    o_ref[...] = jnp.maximum(acc * gam_ref[...] + bet_ref[...], 0.0)


def _finish(wr, vox, gs, beta, interpret=False):
    grid = HW // 128
    return pl.pallas_call(
        _finish_body,
        grid=(grid,),
        in_specs=[
            pl.BlockSpec((BEVZ, BEVC, CCTX), lambda i: (0, 0, 0)),
            pl.BlockSpec((BEVZ, 128, FW), lambda i: (0, i, 0)),
            pl.BlockSpec((BEVC, 1), lambda i: (0, 0)),
            pl.BlockSpec((BEVC, 1), lambda i: (0, 0)),
        ],
        out_specs=pl.BlockSpec((BEVC, 128), lambda i: (0, i)),
        out_shape=jax.ShapeDtypeStruct((BEVC, HW), jnp.float32),
        interpret=interpret,
    )(wr, vox, gs, beta)


def kernel(depth_prob, context, intrinsics, cam2ego, W, gamma, beta):
    vind = _geometry(intrinsics, cam2ego).reshape(-1)
    dp4 = depth_prob.reshape(N, D, PIX, 1)
    ctx3 = jnp.transpose(context.reshape(N, CCTX, PIX), (0, 2, 1))
    feat = _feat(dp4, ctx3)
    vox = _sc_scatter(feat, vind)
    vox3 = vox.reshape(BEVZ, HW, FW)
    wr = jnp.transpose(W.reshape(BEVC, CCTX, BEVZ), (2, 0, 1))
    gs = (gamma / math.sqrt(1.0 + 1e-5)).reshape(BEVC, 1)
    y = _finish(wr, vox3, gs, beta.reshape(BEVC, 1))
    return y.reshape(1, BEVC, BEVH, BEVW)
